# baseline (speedup 1.0000x reference)
import functools

import jax
import jax.numpy as jnp
from jax import lax
from jax.experimental import pallas as pl
from jax.experimental.pallas import tpu as pltpu
from jax.experimental.pallas import tpu_sc as plsc

N = 10000
E = 320000
F_IN = 128
HID = 128
NC = 40

NP = 10240
CH = 128
NTILES = 32
NCHUNK = 80
EPT = NCHUNK * CH
EPAD = NTILES * EPT
RPT = NP // 16


def _copy_idx(src_ref, base, dst_ref):
    for j in range(CH // 16):
        dst_ref[pl.ds(j * 16, 16)] = src_ref[pl.ds(base + j * 16, 16)]


@functools.cache
def _get_sc_deg():
    mesh = plsc.VectorSubcoreMesh(core_axis_name="c", subcore_axis_name="s")

    @functools.partial(
        pl.kernel,
        out_type=jax.ShapeDtypeStruct((2, NP, 128), jnp.float32),
        mesh=mesh,
        scratch_types=[
            pltpu.VMEM((CH,), jnp.int32),
            pltpu.VMEM((CH,), jnp.int32),
            pltpu.VMEM((CH, 128), jnp.float32),
            pltpu.VMEM_SHARED((NP, 128), jnp.float32),
            pltpu.SemaphoreType.DMA,
            pltpu.SemaphoreType.DMA,
            pltpu.SemaphoreType.DMA,
            pltpu.SemaphoreType.DMA,
        ],
    )
    def deg(dst_hbm, ones_hbm, zeros_hbm, out_hbm,
            idxd_a, idxd_b, ones_v, acc_sh, sem_ia, sem_ib, sem_sa, sem_sb):
        c = lax.axis_index("c")
        s = lax.axis_index("s")
        wid = c * 16 + s
        t0 = wid * EPT
        pltpu.sync_copy(ones_hbm, ones_v)
        pltpu.sync_copy(zeros_hbm.at[pl.ds(s * RPT, RPT)],
                        acc_sh.at[pl.ds(s * RPT, RPT)])
        pltpu.async_copy(dst_hbm.at[pl.ds(t0, CH)], idxd_a, sem_ia)
        plsc.subcore_barrier()

        def body(p, carry):
            i1 = t0 + (2 * p + 1) * CH
            i2 = t0 + (2 * p + 2) * CH

            @pl.when(p > 0)
            def _():
                pltpu.make_async_copy(ones_v, acc_sh.at[idxd_b], sem_sb).wait()

            pltpu.async_copy(dst_hbm.at[pl.ds(i1, CH)], idxd_b, sem_ib)
            pltpu.make_async_copy(dst_hbm.at[pl.ds(t0, CH)], idxd_a, sem_ia).wait()
            pltpu.async_copy(ones_v, acc_sh.at[idxd_a], sem_sa, add=True)
            pltpu.make_async_copy(dst_hbm.at[pl.ds(t0, CH)], idxd_b, sem_ib).wait()
            pltpu.async_copy(ones_v, acc_sh.at[idxd_b], sem_sb, add=True)

            @pl.when(2 * p + 2 < NCHUNK)
            def _():
                pltpu.make_async_copy(ones_v, acc_sh.at[idxd_a], sem_sa).wait()
                pltpu.async_copy(dst_hbm.at[pl.ds(i2, CH)], idxd_a, sem_ia)
            return carry

        lax.fori_loop(0, NCHUNK // 2, body, 0)
        pltpu.make_async_copy(ones_v, acc_sh.at[idxd_a], sem_sa).wait()
        pltpu.make_async_copy(ones_v, acc_sh.at[idxd_b], sem_sb).wait()
        plsc.subcore_barrier()
        pltpu.sync_copy(acc_sh.at[pl.ds(s * RPT, RPT)],
                        out_hbm.at[c, pl.ds(s * RPT, RPT)])

    return deg


@functools.cache
def _get_sc_agg(D):
    mesh = plsc.VectorSubcoreMesh(core_axis_name="c", subcore_axis_name="s")

    @functools.partial(
        pl.kernel,
        out_type=jax.ShapeDtypeStruct((2, NP, D), jnp.float32),
        mesh=mesh,
        scratch_types=[
            pltpu.VMEM((EPT,), jnp.int32),
            pltpu.VMEM((CH,), jnp.int32),
            pltpu.VMEM((CH,), jnp.int32),
            pltpu.VMEM((CH, D), jnp.float32),
            pltpu.VMEM((CH, D), jnp.float32),
            pltpu.VMEM_SHARED((NP, D), jnp.float32),
            pltpu.SemaphoreType.DMA,
            pltpu.SemaphoreType.DMA,
            pltpu.SemaphoreType.DMA,
            pltpu.SemaphoreType.DMA,
            pltpu.SemaphoreType.DMA,
            pltpu.SemaphoreType.DMA,
        ],
    )
    def agg(g_hbm, src_hbm, dst_hbm, zeros_hbm, out_hbm,
            srcv, idxd_a, idxd_b, rows_a, rows_b, acc_sh,
            sem_ia, sem_ib, sem_ga, sem_gb, sem_sa, sem_sb):
        c = lax.axis_index("c")
        s = lax.axis_index("s")
        wid = c * 16 + s
        t0 = wid * EPT

        pltpu.sync_copy(src_hbm.at[pl.ds(t0, EPT)], srcv)
        pltpu.sync_copy(zeros_hbm.at[pl.ds(s * RPT, RPT)],
                        acc_sh.at[pl.ds(s * RPT, RPT)])
        pltpu.async_copy(dst_hbm.at[pl.ds(t0, CH)], idxd_a, sem_ia)
        pltpu.async_copy(g_hbm.at[srcv.at[pl.ds(0, CH)]], rows_a, sem_ga)
        plsc.subcore_barrier()

        def body(p, carry):
            i1 = (2 * p + 1) * CH
            i2 = (2 * p + 2) * CH

            @pl.when(p > 0)
            def _():
                pltpu.make_async_copy(rows_b, acc_sh.at[idxd_b], sem_sb).wait()

            pltpu.async_copy(dst_hbm.at[pl.ds(t0 + i1, CH)], idxd_b, sem_ib)
            pltpu.async_copy(g_hbm.at[srcv.at[pl.ds(i1, CH)]], rows_b, sem_gb)
            pltpu.make_async_copy(g_hbm.at[srcv.at[pl.ds(0, CH)]],
                                  rows_a, sem_ga).wait()
            pltpu.make_async_copy(dst_hbm.at[pl.ds(t0, CH)], idxd_a, sem_ia).wait()
            pltpu.async_copy(rows_a, acc_sh.at[idxd_a], sem_sa, add=True)
            pltpu.make_async_copy(g_hbm.at[srcv.at[pl.ds(0, CH)]],
                                  rows_b, sem_gb).wait()
            pltpu.make_async_copy(dst_hbm.at[pl.ds(t0, CH)], idxd_b, sem_ib).wait()
            pltpu.async_copy(rows_b, acc_sh.at[idxd_b], sem_sb, add=True)

            @pl.when(2 * p + 2 < NCHUNK)
            def _():
                pltpu.make_async_copy(rows_a, acc_sh.at[idxd_a], sem_sa).wait()
                pltpu.async_copy(dst_hbm.at[pl.ds(t0 + i2, CH)], idxd_a, sem_ia)
                pltpu.async_copy(g_hbm.at[srcv.at[pl.ds(i2, CH)]], rows_a, sem_ga)
            return carry

        lax.fori_loop(0, NCHUNK // 2, body, 0)
        pltpu.make_async_copy(rows_a, acc_sh.at[idxd_a], sem_sa).wait()
        pltpu.make_async_copy(rows_b, acc_sh.at[idxd_b], sem_sb).wait()
        plsc.subcore_barrier()
        pltpu.sync_copy(acc_sh.at[pl.ds(s * RPT, RPT)],
                        out_hbm.at[c, pl.ds(s * RPT, RPT)])

    return agg


BR = 1024


def _dinv_from(degm):
    deg = degm[0, :, 0:1] + degm[1, :, 0:1] + 1.0
    return lax.rsqrt(deg)


def _tc_pre1_body(x_ref, w_ref, degm_ref, o_ref):
    dinv = _dinv_from(degm_ref[...])
    h = jnp.dot(x_ref[...], w_ref[...], preferred_element_type=jnp.float32)
    o_ref[...] = h * dinv


def _tc_mid_body(part_ref, g_ref, degm_ref, b1_ref, w2_ref, o_ref):
    dinv = _dinv_from(degm_ref[...])
    agg = part_ref[0] + part_ref[1] + g_ref[...]
    h1 = jnp.maximum(agg * dinv + b1_ref[...], 0.0)
    o_ref[...] = jnp.dot(h1, w2_ref[...], preferred_element_type=jnp.float32) * dinv


def _tc_fin_body(part_ref, g_ref, degm_ref, b2_ref, o_ref):
    dinv = _dinv_from(degm_ref[...])
    o = (part_ref[0] + part_ref[1] + g_ref[...]) * dinv + b2_ref[...]
    mask = lax.broadcasted_iota(jnp.int32, o.shape, 1) < NC
    om = jnp.where(mask, o, -1e30)
    m = jnp.max(om, axis=1, keepdims=True)
    ex = jnp.where(mask, jnp.exp(o - m), 0.0)
    lse = jnp.log(jnp.sum(ex, axis=1, keepdims=True))
    o_ref[...] = o - m - lse


_GRID = NP // BR


def _tc_pre1(x_pad, W1, degm):
    return pl.pallas_call(
        _tc_pre1_body,
        grid=(_GRID,),
        in_specs=[
            pl.BlockSpec((BR, F_IN), lambda i: (i, 0)),
            pl.BlockSpec((F_IN, HID), lambda i: (0, 0)),
            pl.BlockSpec((2, BR, 128), lambda i: (0, i, 0)),
        ],
        out_specs=pl.BlockSpec((BR, HID), lambda i: (i, 0)),
        out_shape=jax.ShapeDtypeStruct((NP, HID), jnp.float32),
    )(x_pad, W1, degm)


def _tc_mid(part1, g1, degm, b1, W2p):
    return pl.pallas_call(
        _tc_mid_body,
        grid=(_GRID,),
        in_specs=[
            pl.BlockSpec((2, BR, HID), lambda i: (0, i, 0)),
            pl.BlockSpec((BR, HID), lambda i: (i, 0)),
            pl.BlockSpec((2, BR, 128), lambda i: (0, i, 0)),
            pl.BlockSpec((1, HID), lambda i: (0, 0)),
            pl.BlockSpec((HID, HID), lambda i: (0, 0)),
        ],
        out_specs=pl.BlockSpec((BR, HID), lambda i: (i, 0)),
        out_shape=jax.ShapeDtypeStruct((NP, HID), jnp.float32),
    )(part1, g1, degm, b1, W2p)


def _tc_fin(part2, g2, degm, b2p):
    return pl.pallas_call(
        _tc_fin_body,
        grid=(_GRID,),
        in_specs=[
            pl.BlockSpec((2, BR, HID), lambda i: (0, i, 0)),
            pl.BlockSpec((BR, HID), lambda i: (i, 0)),
            pl.BlockSpec((2, BR, 128), lambda i: (0, i, 0)),
            pl.BlockSpec((1, HID), lambda i: (0, 0)),
        ],
        out_specs=pl.BlockSpec((BR, HID), lambda i: (i, 0)),
        out_shape=jax.ShapeDtypeStruct((NP, HID), jnp.float32),
    )(part2, g2, degm, b2p)


def kernel(x, edge_index, W1, b1, W2, b2):
    x_pad = jnp.zeros((NP, F_IN), jnp.float32).at[:N].set(x)
    npad = EPAD - E
    padi = (N + (jnp.arange(npad, dtype=jnp.int32) % (NP - N))).astype(jnp.int32)
    src = jnp.concatenate([edge_index[0], padi])
    dst = jnp.concatenate([edge_index[1], padi])

    W2p = jnp.zeros((HID, HID), jnp.float32).at[:, :NC].set(W2)
    b1r = b1.reshape(1, HID)
    b2p = jnp.zeros((1, HID), jnp.float32).at[0, :NC].set(b2)

    ones128 = jnp.ones((CH, 128), jnp.float32)
    z128 = jnp.zeros((NP, HID), jnp.float32)

    degm = _get_sc_deg()(dst, ones128, z128)
    g1 = _tc_pre1(x_pad, W1, degm)
    part1 = _get_sc_agg(HID)(g1, src, dst, z128)
    g2 = _tc_mid(part1, g1, degm, b1r, W2p)
    part2 = _get_sc_agg(HID)(g2, src, dst, z128)
    out = _tc_fin(part2, g2, degm, b2p)
    return out[:N, :NC]


# speedup vs baseline: 1.2191x; 1.2191x over previous
import functools

import jax
import jax.numpy as jnp
from jax import lax
from jax.experimental import pallas as pl
from jax.experimental.pallas import tpu as pltpu
from jax.experimental.pallas import tpu_sc as plsc

N = 10000
E = 320000
F_IN = 128
HID = 128
NC = 40

NP = 10240
CH = 128
NTILES = 32
NCHUNK = 80
EPT = NCHUNK * CH
EPAD = NTILES * EPT
RPT = NP // 16


def _copy_idx(src_ref, base, dst_ref):
    for j in range(CH // 16):
        dst_ref[pl.ds(j * 16, 16)] = src_ref[pl.ds(base + j * 16, 16)]


@functools.cache
def _get_sc_deg():
    mesh = plsc.VectorSubcoreMesh(core_axis_name="c", subcore_axis_name="s")

    @functools.partial(
        pl.kernel,
        out_type=jax.ShapeDtypeStruct((2, NP, 128), jnp.float32),
        mesh=mesh,
        scratch_types=[
            pltpu.VMEM((CH,), jnp.int32),
            pltpu.VMEM((CH,), jnp.int32),
            pltpu.VMEM((CH, 128), jnp.float32),
            pltpu.VMEM_SHARED((NP, 128), jnp.float32),
            pltpu.SemaphoreType.DMA,
            pltpu.SemaphoreType.DMA,
            pltpu.SemaphoreType.DMA,
            pltpu.SemaphoreType.DMA,
        ],
    )
    def deg(dst_hbm, ones_hbm, zeros_hbm, out_hbm,
            idxd_a, idxd_b, ones_v, acc_sh, sem_ia, sem_ib, sem_sa, sem_sb):
        c = lax.axis_index("c")
        s = lax.axis_index("s")
        wid = c * 16 + s
        t0 = wid * EPT
        pltpu.sync_copy(ones_hbm, ones_v)
        pltpu.sync_copy(zeros_hbm.at[pl.ds(s * RPT, RPT)],
                        acc_sh.at[pl.ds(s * RPT, RPT)])
        pltpu.async_copy(dst_hbm.at[pl.ds(t0, CH)], idxd_a, sem_ia)
        plsc.subcore_barrier()

        def body(p, carry):
            i1 = t0 + (2 * p + 1) * CH
            i2 = t0 + (2 * p + 2) * CH

            @pl.when(p > 0)
            def _():
                pltpu.make_async_copy(ones_v, acc_sh.at[idxd_b], sem_sb).wait()

            pltpu.async_copy(dst_hbm.at[pl.ds(i1, CH)], idxd_b, sem_ib)
            pltpu.make_async_copy(dst_hbm.at[pl.ds(t0, CH)], idxd_a, sem_ia).wait()
            pltpu.async_copy(ones_v, acc_sh.at[idxd_a], sem_sa, add=True)

            @pl.when(2 * p + 2 < NCHUNK)
            def _():
                pltpu.make_async_copy(ones_v, acc_sh.at[idxd_a], sem_sa).wait()
                pltpu.async_copy(dst_hbm.at[pl.ds(i2, CH)], idxd_a, sem_ia)

            pltpu.make_async_copy(dst_hbm.at[pl.ds(t0, CH)], idxd_b, sem_ib).wait()
            pltpu.async_copy(ones_v, acc_sh.at[idxd_b], sem_sb, add=True)
            return carry

        lax.fori_loop(0, NCHUNK // 2, body, 0)
        pltpu.make_async_copy(ones_v, acc_sh.at[idxd_a], sem_sa).wait()
        pltpu.make_async_copy(ones_v, acc_sh.at[idxd_b], sem_sb).wait()
        plsc.subcore_barrier()
        pltpu.sync_copy(acc_sh.at[pl.ds(s * RPT, RPT)],
                        out_hbm.at[c, pl.ds(s * RPT, RPT)])

    return deg


@functools.cache
def _get_sc_agg(D):
    mesh = plsc.VectorSubcoreMesh(core_axis_name="c", subcore_axis_name="s")

    @functools.partial(
        pl.kernel,
        out_type=jax.ShapeDtypeStruct((2, NP, D), jnp.float32),
        mesh=mesh,
        scratch_types=[
            pltpu.VMEM((EPT,), jnp.int32),
            pltpu.VMEM((CH,), jnp.int32),
            pltpu.VMEM((CH,), jnp.int32),
            pltpu.VMEM((CH, D), jnp.float32),
            pltpu.VMEM((CH, D), jnp.float32),
            pltpu.VMEM_SHARED((NP, D), jnp.float32),
            pltpu.SemaphoreType.DMA,
            pltpu.SemaphoreType.DMA,
            pltpu.SemaphoreType.DMA,
            pltpu.SemaphoreType.DMA,
            pltpu.SemaphoreType.DMA,
            pltpu.SemaphoreType.DMA,
        ],
    )
    def agg(g_hbm, src_hbm, dst_hbm, zeros_hbm, out_hbm,
            srcv, idxd_a, idxd_b, rows_a, rows_b, acc_sh,
            sem_ia, sem_ib, sem_ga, sem_gb, sem_sa, sem_sb):
        c = lax.axis_index("c")
        s = lax.axis_index("s")
        wid = c * 16 + s
        t0 = wid * EPT

        pltpu.sync_copy(src_hbm.at[pl.ds(t0, EPT)], srcv)
        pltpu.sync_copy(zeros_hbm.at[pl.ds(s * RPT, RPT)],
                        acc_sh.at[pl.ds(s * RPT, RPT)])
        pltpu.async_copy(dst_hbm.at[pl.ds(t0, CH)], idxd_a, sem_ia)
        pltpu.async_copy(g_hbm.at[srcv.at[pl.ds(0, CH)]], rows_a, sem_ga)
        plsc.subcore_barrier()

        def body(p, carry):
            i1 = (2 * p + 1) * CH
            i2 = (2 * p + 2) * CH

            @pl.when(p > 0)
            def _():
                pltpu.make_async_copy(rows_b, acc_sh.at[idxd_b], sem_sb).wait()

            pltpu.async_copy(dst_hbm.at[pl.ds(t0 + i1, CH)], idxd_b, sem_ib)
            pltpu.async_copy(g_hbm.at[srcv.at[pl.ds(i1, CH)]], rows_b, sem_gb)
            pltpu.make_async_copy(g_hbm.at[srcv.at[pl.ds(0, CH)]],
                                  rows_a, sem_ga).wait()
            pltpu.make_async_copy(dst_hbm.at[pl.ds(t0, CH)], idxd_a, sem_ia).wait()
            pltpu.async_copy(rows_a, acc_sh.at[idxd_a], sem_sa, add=True)

            @pl.when(2 * p + 2 < NCHUNK)
            def _():
                pltpu.make_async_copy(rows_a, acc_sh.at[idxd_a], sem_sa).wait()
                pltpu.async_copy(dst_hbm.at[pl.ds(t0 + i2, CH)], idxd_a, sem_ia)
                pltpu.async_copy(g_hbm.at[srcv.at[pl.ds(i2, CH)]], rows_a, sem_ga)

            pltpu.make_async_copy(g_hbm.at[srcv.at[pl.ds(0, CH)]],
                                  rows_b, sem_gb).wait()
            pltpu.make_async_copy(dst_hbm.at[pl.ds(t0, CH)], idxd_b, sem_ib).wait()
            pltpu.async_copy(rows_b, acc_sh.at[idxd_b], sem_sb, add=True)
            return carry

        lax.fori_loop(0, NCHUNK // 2, body, 0)
        pltpu.make_async_copy(rows_a, acc_sh.at[idxd_a], sem_sa).wait()
        pltpu.make_async_copy(rows_b, acc_sh.at[idxd_b], sem_sb).wait()
        plsc.subcore_barrier()
        pltpu.sync_copy(acc_sh.at[pl.ds(s * RPT, RPT)],
                        out_hbm.at[c, pl.ds(s * RPT, RPT)])

    return agg


BR = 1024


def _dinv_from(degm):
    deg = degm[0, :, 0:1] + degm[1, :, 0:1] + 1.0
    return lax.rsqrt(deg)


def _tc_pre1_body(x_ref, w_ref, degm_ref, o_ref):
    dinv = _dinv_from(degm_ref[...])
    h = jnp.dot(x_ref[...], w_ref[...], preferred_element_type=jnp.float32)
    o_ref[...] = h * dinv


def _tc_mid_body(part_ref, g_ref, degm_ref, b1_ref, w2_ref, o_ref):
    dinv = _dinv_from(degm_ref[...])
    agg = part_ref[0] + part_ref[1] + g_ref[...]
    h1 = jnp.maximum(agg * dinv + b1_ref[...], 0.0)
    o_ref[...] = jnp.dot(h1, w2_ref[...], preferred_element_type=jnp.float32) * dinv


def _tc_fin_body(part_ref, g_ref, degm_ref, b2_ref, o_ref):
    dinv = _dinv_from(degm_ref[...])
    o = (part_ref[0] + part_ref[1] + g_ref[...]) * dinv + b2_ref[...]
    mask = lax.broadcasted_iota(jnp.int32, o.shape, 1) < NC
    om = jnp.where(mask, o, -1e30)
    m = jnp.max(om, axis=1, keepdims=True)
    ex = jnp.where(mask, jnp.exp(o - m), 0.0)
    lse = jnp.log(jnp.sum(ex, axis=1, keepdims=True))
    o_ref[...] = o - m - lse


_GRID = NP // BR


def _tc_pre1(x_pad, W1, degm):
    return pl.pallas_call(
        _tc_pre1_body,
        grid=(_GRID,),
        in_specs=[
            pl.BlockSpec((BR, F_IN), lambda i: (i, 0)),
            pl.BlockSpec((F_IN, HID), lambda i: (0, 0)),
            pl.BlockSpec((2, BR, 128), lambda i: (0, i, 0)),
        ],
        out_specs=pl.BlockSpec((BR, HID), lambda i: (i, 0)),
        out_shape=jax.ShapeDtypeStruct((NP, HID), jnp.float32),
    )(x_pad, W1, degm)


def _tc_mid(part1, g1, degm, b1, W2p):
    return pl.pallas_call(
        _tc_mid_body,
        grid=(_GRID,),
        in_specs=[
            pl.BlockSpec((2, BR, HID), lambda i: (0, i, 0)),
            pl.BlockSpec((BR, HID), lambda i: (i, 0)),
            pl.BlockSpec((2, BR, 128), lambda i: (0, i, 0)),
            pl.BlockSpec((1, HID), lambda i: (0, 0)),
            pl.BlockSpec((HID, HID), lambda i: (0, 0)),
        ],
        out_specs=pl.BlockSpec((BR, HID), lambda i: (i, 0)),
        out_shape=jax.ShapeDtypeStruct((NP, HID), jnp.float32),
    )(part1, g1, degm, b1, W2p)


def _tc_fin(part2, g2, degm, b2p):
    return pl.pallas_call(
        _tc_fin_body,
        grid=(_GRID,),
        in_specs=[
            pl.BlockSpec((2, BR, HID), lambda i: (0, i, 0)),
            pl.BlockSpec((BR, HID), lambda i: (i, 0)),
            pl.BlockSpec((2, BR, 128), lambda i: (0, i, 0)),
            pl.BlockSpec((1, HID), lambda i: (0, 0)),
        ],
        out_specs=pl.BlockSpec((BR, HID), lambda i: (i, 0)),
        out_shape=jax.ShapeDtypeStruct((NP, HID), jnp.float32),
    )(part2, g2, degm, b2p)


def kernel(x, edge_index, W1, b1, W2, b2):
    x_pad = jnp.zeros((NP, F_IN), jnp.float32).at[:N].set(x)
    npad = EPAD - E
    padi = (N + (jnp.arange(npad, dtype=jnp.int32) % (NP - N))).astype(jnp.int32)
    src = jnp.concatenate([edge_index[0], padi])
    dst = jnp.concatenate([edge_index[1], padi])

    W2p = jnp.zeros((HID, HID), jnp.float32).at[:, :NC].set(W2)
    b1r = b1.reshape(1, HID)
    b2p = jnp.zeros((1, HID), jnp.float32).at[0, :NC].set(b2)

    ones128 = jnp.ones((CH, 128), jnp.float32)
    z128 = jnp.zeros((NP, HID), jnp.float32)

    degm = _get_sc_deg()(dst, ones128, z128)
    g1 = _tc_pre1(x_pad, W1, degm)
    part1 = _get_sc_agg(HID)(g1, src, dst, z128)
    g2 = _tc_mid(part1, g1, degm, b1r, W2p)
    part2 = _get_sc_agg(HID)(g2, src, dst, z128)
    out = _tc_fin(part2, g2, degm, b2p)
    return out[:N, :NC]


# speedup vs baseline: 1.2192x; 1.0001x over previous
import functools

import jax
import jax.numpy as jnp
from jax import lax
from jax.experimental import pallas as pl
from jax.experimental.pallas import tpu as pltpu
from jax.experimental.pallas import tpu_sc as plsc

N = 10000
E = 320000
F_IN = 128
HID = 128
NC = 40

NP = 10240
CH = 128
NTILES = 32
NCHUNK = 80
EPT = NCHUNK * CH
EPAD = NTILES * EPT
RPT = NP // 16


def _copy_idx(src_ref, base, dst_ref):
    for j in range(CH // 16):
        dst_ref[pl.ds(j * 16, 16)] = src_ref[pl.ds(base + j * 16, 16)]


@functools.cache
def _get_sc_deg():
    mesh = plsc.VectorSubcoreMesh(core_axis_name="c", subcore_axis_name="s")

    @functools.partial(
        pl.kernel,
        out_type=jax.ShapeDtypeStruct((2, NP, 128), jnp.float32),
        mesh=mesh,
        scratch_types=[
            pltpu.VMEM((CH,), jnp.int32),
            pltpu.VMEM((CH,), jnp.int32),
            pltpu.VMEM((CH, 128), jnp.float32),
            pltpu.VMEM_SHARED((NP, 128), jnp.float32),
            pltpu.SemaphoreType.DMA,
            pltpu.SemaphoreType.DMA,
            pltpu.SemaphoreType.DMA,
            pltpu.SemaphoreType.DMA,
        ],
    )
    def deg(dst_hbm, ones_hbm, zeros_hbm, out_hbm,
            idxd_a, idxd_b, ones_v, acc_sh, sem_ia, sem_ib, sem_sa, sem_sb):
        c = lax.axis_index("c")
        s = lax.axis_index("s")
        wid = c * 16 + s
        t0 = wid * EPT
        pltpu.sync_copy(ones_hbm, ones_v)
        pltpu.sync_copy(zeros_hbm.at[pl.ds(s * RPT, RPT)],
                        acc_sh.at[pl.ds(s * RPT, RPT)])
        pltpu.async_copy(dst_hbm.at[pl.ds(t0, CH)], idxd_a, sem_ia)
        plsc.subcore_barrier()

        def body(p, carry):
            i1 = t0 + (2 * p + 1) * CH
            i2 = t0 + (2 * p + 2) * CH

            @pl.when(p > 0)
            def _():
                pltpu.make_async_copy(ones_v, acc_sh.at[idxd_b], sem_sb).wait()

            pltpu.async_copy(dst_hbm.at[pl.ds(i1, CH)], idxd_b, sem_ib)
            pltpu.make_async_copy(dst_hbm.at[pl.ds(t0, CH)], idxd_a, sem_ia).wait()
            pltpu.async_copy(ones_v, acc_sh.at[idxd_a], sem_sa, add=True)

            @pl.when(2 * p + 2 < NCHUNK)
            def _():
                pltpu.make_async_copy(ones_v, acc_sh.at[idxd_a], sem_sa).wait()
                pltpu.async_copy(dst_hbm.at[pl.ds(i2, CH)], idxd_a, sem_ia)

            pltpu.make_async_copy(dst_hbm.at[pl.ds(t0, CH)], idxd_b, sem_ib).wait()
            pltpu.async_copy(ones_v, acc_sh.at[idxd_b], sem_sb, add=True)
            return carry

        lax.fori_loop(0, NCHUNK // 2, body, 0)
        pltpu.make_async_copy(ones_v, acc_sh.at[idxd_a], sem_sa).wait()
        pltpu.make_async_copy(ones_v, acc_sh.at[idxd_b], sem_sb).wait()
        plsc.subcore_barrier()
        pltpu.sync_copy(acc_sh.at[pl.ds(s * RPT, RPT)],
                        out_hbm.at[c, pl.ds(s * RPT, RPT)])

    return deg


@functools.cache
def _get_sc_agg(D):
    mesh = plsc.VectorSubcoreMesh(core_axis_name="c", subcore_axis_name="s")

    @functools.partial(
        pl.kernel,
        out_type=jax.ShapeDtypeStruct((2, NP, D), jnp.float32),
        mesh=mesh,
        scratch_types=[
            pltpu.VMEM((EPT,), jnp.int32),
            pltpu.VMEM((CH,), jnp.int32),
            pltpu.VMEM((CH,), jnp.int32),
            pltpu.VMEM((CH, D), jnp.float32),
            pltpu.VMEM((CH, D), jnp.float32),
            pltpu.VMEM_SHARED((NP, D), jnp.float32),
            pltpu.SemaphoreType.DMA,
            pltpu.SemaphoreType.DMA,
            pltpu.SemaphoreType.DMA,
            pltpu.SemaphoreType.DMA,
            pltpu.SemaphoreType.DMA,
            pltpu.SemaphoreType.DMA,
        ],
    )
    def agg(g_hbm, src_hbm, dst_hbm, zeros_hbm, out_hbm,
            srcv, idxd_a, idxd_b, rows_a, rows_b, acc_sh,
            sem_ia, sem_ib, sem_ga, sem_gb, sem_sa, sem_sb):
        c = lax.axis_index("c")
        s = lax.axis_index("s")
        wid = c * 16 + s
        t0 = wid * EPT

        pltpu.sync_copy(src_hbm.at[pl.ds(t0, EPT)], srcv)
        pltpu.sync_copy(zeros_hbm.at[pl.ds(s * RPT, RPT)],
                        acc_sh.at[pl.ds(s * RPT, RPT)])
        pltpu.async_copy(dst_hbm.at[pl.ds(t0, CH)], idxd_a, sem_ia)
        pltpu.async_copy(g_hbm.at[srcv.at[pl.ds(0, CH)]], rows_a, sem_ga)
        plsc.subcore_barrier()

        def body(p, carry):
            i1 = (2 * p + 1) * CH
            i2 = (2 * p + 2) * CH

            @pl.when(p > 0)
            def _():
                pltpu.make_async_copy(rows_b, acc_sh.at[idxd_b], sem_sb).wait()

            pltpu.async_copy(dst_hbm.at[pl.ds(t0 + i1, CH)], idxd_b, sem_ib)
            pltpu.async_copy(g_hbm.at[srcv.at[pl.ds(i1, CH)]], rows_b, sem_gb)
            pltpu.make_async_copy(g_hbm.at[srcv.at[pl.ds(0, CH)]],
                                  rows_a, sem_ga).wait()
            pltpu.make_async_copy(dst_hbm.at[pl.ds(t0, CH)], idxd_a, sem_ia).wait()
            pltpu.async_copy(rows_a, acc_sh.at[idxd_a], sem_sa, add=True)

            @pl.when(2 * p + 2 < NCHUNK)
            def _():
                pltpu.make_async_copy(rows_a, acc_sh.at[idxd_a], sem_sa).wait()
                pltpu.async_copy(dst_hbm.at[pl.ds(t0 + i2, CH)], idxd_a, sem_ia)
                pltpu.async_copy(g_hbm.at[srcv.at[pl.ds(i2, CH)]], rows_a, sem_ga)

            pltpu.make_async_copy(g_hbm.at[srcv.at[pl.ds(0, CH)]],
                                  rows_b, sem_gb).wait()
            pltpu.make_async_copy(dst_hbm.at[pl.ds(t0, CH)], idxd_b, sem_ib).wait()
            pltpu.async_copy(rows_b, acc_sh.at[idxd_b], sem_sb, add=True)
            return carry

        lax.fori_loop(0, NCHUNK // 2, body, 0)
        pltpu.make_async_copy(rows_a, acc_sh.at[idxd_a], sem_sa).wait()
        pltpu.make_async_copy(rows_b, acc_sh.at[idxd_b], sem_sb).wait()
        plsc.subcore_barrier()
        pltpu.sync_copy(acc_sh.at[pl.ds(s * RPT, RPT)],
                        out_hbm.at[c, pl.ds(s * RPT, RPT)])

    return agg


BR = 1024


def _dinv_from(degm):
    deg = degm[0, :, 0:1] + degm[1, :, 0:1] + 1.0
    return lax.rsqrt(deg)


def _tc_pre1_body(x_ref, w_ref, degm_ref, o_ref, d8_ref):
    dinv = _dinv_from(degm_ref[...])
    h = jnp.dot(x_ref[...], w_ref[...], preferred_element_type=jnp.float32)
    o_ref[...] = h * dinv
    d8_ref[...] = jnp.broadcast_to(dinv, (dinv.shape[0], 8))


def _tc_mid_body(part_ref, g_ref, d8_ref, b1_ref, w2_ref, o_ref):
    dinv = d8_ref[:, 0:1]
    agg = part_ref[0] + part_ref[1] + g_ref[...]
    h1 = jnp.maximum(agg * dinv + b1_ref[...], 0.0)
    o_ref[...] = jnp.dot(h1, w2_ref[...], preferred_element_type=jnp.float32) * dinv


def _tc_fin_body(part_ref, g_ref, d8_ref, b2_ref, o_ref):
    dinv = d8_ref[:, 0:1]
    o = (part_ref[0] + part_ref[1] + g_ref[...]) * dinv + b2_ref[...]
    mask = lax.broadcasted_iota(jnp.int32, o.shape, 1) < NC
    om = jnp.where(mask, o, -1e30)
    m = jnp.max(om, axis=1, keepdims=True)
    ex = jnp.where(mask, jnp.exp(o - m), 0.0)
    lse = jnp.log(jnp.sum(ex, axis=1, keepdims=True))
    o_ref[...] = o - m - lse


_GRID = NP // BR


def _tc_pre1(x_pad, W1, degm):
    return pl.pallas_call(
        _tc_pre1_body,
        grid=(_GRID,),
        in_specs=[
            pl.BlockSpec((BR, F_IN), lambda i: (i, 0)),
            pl.BlockSpec((F_IN, HID), lambda i: (0, 0)),
            pl.BlockSpec((2, BR, 128), lambda i: (0, i, 0)),
        ],
        out_specs=[
            pl.BlockSpec((BR, HID), lambda i: (i, 0)),
            pl.BlockSpec((BR, 8), lambda i: (i, 0)),
        ],
        out_shape=[
            jax.ShapeDtypeStruct((NP, HID), jnp.float32),
            jax.ShapeDtypeStruct((NP, 8), jnp.float32),
        ],
    )(x_pad, W1, degm)


def _tc_mid(part1, g1, degm, b1, W2p):
    return pl.pallas_call(
        _tc_mid_body,
        grid=(_GRID,),
        in_specs=[
            pl.BlockSpec((2, BR, HID), lambda i: (0, i, 0)),
            pl.BlockSpec((BR, HID), lambda i: (i, 0)),
            pl.BlockSpec((BR, 8), lambda i: (i, 0)),
            pl.BlockSpec((1, HID), lambda i: (0, 0)),
            pl.BlockSpec((HID, HID), lambda i: (0, 0)),
        ],
        out_specs=pl.BlockSpec((BR, HID), lambda i: (i, 0)),
        out_shape=jax.ShapeDtypeStruct((NP, HID), jnp.float32),
    )(part1, g1, degm, b1, W2p)


def _tc_fin(part2, g2, degm, b2p):
    return pl.pallas_call(
        _tc_fin_body,
        grid=(_GRID,),
        in_specs=[
            pl.BlockSpec((2, BR, HID), lambda i: (0, i, 0)),
            pl.BlockSpec((BR, HID), lambda i: (i, 0)),
            pl.BlockSpec((BR, 8), lambda i: (i, 0)),
            pl.BlockSpec((1, HID), lambda i: (0, 0)),
        ],
        out_specs=pl.BlockSpec((BR, HID), lambda i: (i, 0)),
        out_shape=jax.ShapeDtypeStruct((NP, HID), jnp.float32),
    )(part2, g2, degm, b2p)


def kernel(x, edge_index, W1, b1, W2, b2):
    x_pad = jnp.zeros((NP, F_IN), jnp.float32).at[:N].set(x)
    npad = EPAD - E
    padi = (N + (jnp.arange(npad, dtype=jnp.int32) % (NP - N))).astype(jnp.int32)
    src = jnp.concatenate([edge_index[0], padi])
    dst = jnp.concatenate([edge_index[1], padi])

    W2p = jnp.zeros((HID, HID), jnp.float32).at[:, :NC].set(W2)
    b1r = b1.reshape(1, HID)
    b2p = jnp.zeros((1, HID), jnp.float32).at[0, :NC].set(b2)

    ones128 = jnp.ones((CH, 128), jnp.float32)
    z128 = jnp.zeros((NP, HID), jnp.float32)

    degm = _get_sc_deg()(dst, ones128, z128)
    g1, d8 = _tc_pre1(x_pad, W1, degm)
    part1 = _get_sc_agg(HID)(g1, src, dst, z128)
    g2 = _tc_mid(part1, g1, d8, b1r, W2p)
    part2 = _get_sc_agg(HID)(g2, src, dst, z128)
    out = _tc_fin(part2, g2, d8, b2p)
    return out[:N, :NC]
